# baseline (speedup 1.0000x reference)
import functools

import jax
import jax.numpy as jnp
import numpy as np
from jax import lax
from jax.experimental import pallas as pl
from jax.experimental.pallas import tpu as pltpu
from jax.experimental.pallas import tpu_sc as plsc

N = 8192
D = 1024
H = 2048
A = 64

NC = 2
NS = 16
NW = NC * NS
L = 16
RPT = N // NW
JPT = RPT // L
TCBLK = 256
CH = 32
NCHUNK = RPT // CH

_LOG2PI = float(np.log(2.0 * np.pi))

_MESH = plsc.VectorSubcoreMesh(core_axis_name="c", subcore_axis_name="s")


def _wid():
    return lax.axis_index("s") * NC + lax.axis_index("c")


def _mask01(v):
    return jnp.minimum(jnp.maximum(v, 0), 1)


def _scan16(x, buf2):
    y = x
    for d in (1, 2, 4, 8):
        buf2[pl.ds(L, L)] = y
        y = y + buf2[pl.ds(L - d, L)]
    return y


@functools.partial(
    pl.kernel, mesh=_MESH,
    out_type=jax.ShapeDtypeStruct((NW * L,), jnp.int32),
    scratch_types=[pltpu.VMEM((RPT,), jnp.int32), pltpu.VMEM((L,), jnp.int32)],
)
def _sc_count(stage_hbm, counts_hbm, svm, cbuf):
    wid = _wid()
    pltpu.sync_copy(stage_hbm.at[pl.ds(wid * RPT, RPT)], svm)
    k = jnp.zeros((L,), jnp.int32)
    for j in range(JPT):
        k = k + _mask01(svm[pl.ds(j * L, L)])
    cbuf[...] = k
    pltpu.sync_copy(cbuf, counts_hbm.at[pl.ds(wid * L, L)])


@functools.partial(
    pl.kernel, mesh=_MESH,
    out_type=[jax.ShapeDtypeStruct((N,), jnp.int32),
              jax.ShapeDtypeStruct((L,), jnp.int32)],
    scratch_types=[
        pltpu.VMEM((RPT,), jnp.int32),
        pltpu.VMEM((NW * L,), jnp.int32),
        pltpu.VMEM((NW * L,), jnp.int32),
        pltpu.VMEM((2 * L,), jnp.int32),
        pltpu.VMEM((2, 128), jnp.int32),
        pltpu.VMEM((2, 128), jnp.int32),
        pltpu.VMEM((L,), jnp.int32),
        pltpu.SemaphoreType.DMA,
    ],
)
def _sc_partition(stage_hbm, counts_hbm, perm_hbm, kout_hbm,
                  svm, cvm, offs_vm, buf2, rbuf, pbuf, kbuf, sem):
    wid = _wid()
    pltpu.sync_copy(stage_hbm.at[pl.ds(wid * RPT, RPT)], svm)
    pltpu.sync_copy(counts_hbm, cvm)
    buf2[pl.ds(0, L)] = jnp.zeros((L,), jnp.int32)

    run = jnp.int32(0)
    for b in range(NW):
        v = cvm[pl.ds(b * L, L)]
        s = _scan16(v, buf2)
        offs_vm[pl.ds(b * L, L)] = s - v + run
        buf2[pl.ds(L, L)] = s
        s_ld = buf2[pl.ds(L, L)]
        run = run + s_ld[L - 1]
    k_total = run

    iota = lax.iota(jnp.int32, L)
    ex = offs_vm[pl.ds(wid * L, L)]
    lane_idx = (wid * L + iota) * JPT
    iex = k_total + lane_idx - ex

    arun = jnp.zeros((L,), jnp.int32)
    for j in range(JPT):
        v = svm[pl.ds(j * L, L)]
        mi = _mask01(v)
        pos_i = iex + (j - arun)
        pos_a = ex + arun
        pos = pos_i + mi * (pos_a - pos_i)
        rid = (wid * RPT + j * L) + iota
        rbuf[j // 8, pl.ds((j % 8) * L, L)] = rid
        pbuf[j // 8, pl.ds((j % 8) * L, L)] = pos
        arun = arun + mi
    h0 = pltpu.async_copy(rbuf.at[0], perm_hbm.at[pbuf.at[0]], sem)
    h1 = pltpu.async_copy(rbuf.at[1], perm_hbm.at[pbuf.at[1]], sem)
    h0.wait()
    h1.wait()

    @pl.when(wid == 0)
    def _():
        kbuf[...] = jnp.zeros((L,), jnp.int32) + k_total
        pltpu.sync_copy(kbuf, kout_hbm)


@functools.partial(
    pl.kernel, mesh=_MESH,
    out_type=[jax.ShapeDtypeStruct((N, D), jnp.float32),
              jax.ShapeDtypeStruct((N, 2 * A), jnp.float32)],
    scratch_types=[
        pltpu.VMEM((RPT,), jnp.int32),
        pltpu.VMEM((CH, D), jnp.float32),
        pltpu.VMEM((CH, D), jnp.float32),
        pltpu.VMEM((CH, 2 * A), jnp.float32),
        pltpu.VMEM((CH, 2 * A), jnp.float32),
        pltpu.VMEM((L,), jnp.int32),
        pltpu.SemaphoreType.DMA,
        pltpu.SemaphoreType.DMA,
    ],
)
def _sc_gather(x_hbm, act_hbm, perm_hbm, kq_hbm, xg_hbm, actg_hbm,
               idxv, xb0, xb1, ab0, ab1, kvm, sem0, sem1):
    wid = _wid()
    pltpu.sync_copy(kq_hbm, kvm)
    kv = kvm[...]
    k_total = kv[0]
    nblk = (k_total + (TCBLK - 1)) // TCBLK
    rows_needed = nblk * TCBLK

    @pl.when(wid * RPT < rows_needed)
    def _():
        pltpu.sync_copy(perm_hbm.at[pl.ds(wid * RPT, RPT)], idxv)
        wb = []
        for c in range(NCHUNK):
            xb = xb0 if c % 2 == 0 else xb1
            ab = ab0 if c % 2 == 0 else ab1
            sem = sem0 if c % 2 == 0 else sem1
            if c >= 2:
                wb[2 * (c - 2)].wait()
                wb[2 * (c - 2) + 1].wait()
            idx_c = idxv.at[pl.ds(c * CH, CH)]
            hx = pltpu.async_copy(x_hbm.at[idx_c], xb, sem)
            ha = pltpu.async_copy(act_hbm.at[idx_c], ab, sem)
            hx.wait()
            ha.wait()
            row0 = wid * RPT + c * CH
            wb.append(pltpu.async_copy(xb, xg_hbm.at[pl.ds(row0, CH)], sem))
            wb.append(pltpu.async_copy(ab, actg_hbm.at[pl.ds(row0, CH)], sem))
        for hdl in wb[-4:]:
            hdl.wait()


@functools.partial(
    pl.kernel, mesh=_MESH,
    out_type=[jax.ShapeDtypeStruct((N,), jnp.float32),
              jax.ShapeDtypeStruct((N,), jnp.float32)],
    scratch_types=[
        pltpu.VMEM((RPT,), jnp.int32),
        pltpu.VMEM((RPT,), jnp.float32),
        pltpu.VMEM((RPT,), jnp.float32),
        pltpu.VMEM((2, 128), jnp.int32),
        pltpu.VMEM((2, 128), jnp.float32),
        pltpu.VMEM((2, 128), jnp.float32),
        pltpu.VMEM((L,), jnp.int32),
        pltpu.SemaphoreType.DMA,
    ],
)
def _sc_scatter(perm_hbm, valg_hbm, logpg_hbm, kq_hbm, val_hbm, logp_hbm,
                pvm, vvm, lvm, pbuf, vbuf, lbuf, kvm, sem):
    wid = _wid()
    pltpu.sync_copy(perm_hbm.at[pl.ds(wid * RPT, RPT)], pvm)
    pltpu.sync_copy(valg_hbm.at[pl.ds(wid * RPT, RPT)], vvm)
    pltpu.sync_copy(logpg_hbm.at[pl.ds(wid * RPT, RPT)], lvm)
    pltpu.sync_copy(kq_hbm, kvm)
    k_vec = kvm[...]
    iota = lax.iota(jnp.int32, L)
    for j in range(JPT):
        gidx = (wid * RPT + j * L) + iota
        live = _mask01(k_vec - gidx).astype(jnp.float32)
        r, c = j // 8, (j % 8) * L
        pbuf[r, pl.ds(c, L)] = pvm[pl.ds(j * L, L)]
        vbuf[r, pl.ds(c, L)] = vvm[pl.ds(j * L, L)] * live
        lbuf[r, pl.ds(c, L)] = lvm[pl.ds(j * L, L)] * live
    handles = [
        pltpu.async_copy(vbuf.at[0], val_hbm.at[pbuf.at[0]], sem),
        pltpu.async_copy(vbuf.at[1], val_hbm.at[pbuf.at[1]], sem),
        pltpu.async_copy(lbuf.at[0], logp_hbm.at[pbuf.at[0]], sem),
        pltpu.async_copy(lbuf.at[1], logp_hbm.at[pbuf.at[1]], sem),
    ]
    for hdl in handles:
        hdl.wait()


def _tc_compact(s_ref, xg_ref, actg_ref, w1_ref, b1_ref, w2_ref, b2_ref,
                logstd_ref, valg_ref, logpg_ref, ent_ref):
    i = pl.program_id(0)
    log_std = logstd_ref[...]
    sum_log_std = jnp.sum(log_std)

    @pl.when(i < s_ref[0])
    def _():
        h = jnp.tanh(jnp.dot(xg_ref[...], w1_ref[...],
                             preferred_element_type=jnp.float32) + b1_ref[...])
        out2 = (jnp.dot(h, w2_ref[...], preferred_element_type=jnp.float32)
                + b2_ref[...])
        val = out2[:, 0:1]
        mu = out2[:, 1:1 + A]
        inv_std = jnp.exp(-log_std)
        diff = (actg_ref[:, 0:A] - mu) * inv_std
        valg_ref[...] = val
        logpg_ref[...] = (-0.5 * jnp.sum(diff * diff, axis=-1, keepdims=True)
                          - sum_log_std - 0.5 * A * _LOG2PI)

    @pl.when(i >= s_ref[0])
    def _():
        valg_ref[...] = jnp.zeros((TCBLK, 1), jnp.float32)
        logpg_ref[...] = jnp.zeros((TCBLK, 1), jnp.float32)

    @pl.when(i == 0)
    def _():
        ent_scalar = sum_log_std + 0.5 * A * (_LOG2PI + 1.0)
        ent_ref[...] = (s_ref[1].astype(jnp.float32)
                        * (ent_scalar / N)).reshape(1, 1)


def _clamped(i, s):
    return (jnp.maximum(jnp.minimum(i, s[0] - 1), 0), 0)


def kernel(stage, x, act, W1, b1, Wv, bv, Wa, ba, log_std):
    stage_i = stage.astype(jnp.int32)
    counts = _sc_count(stage_i)
    perm, kout = _sc_partition(stage_i, counts)
    act128 = jnp.pad(act, ((0, 0), (0, A)))
    xg, actg = _sc_gather(x, act128, perm, kout)

    k_total = kout[0]
    nblk = (k_total + (TCBLK - 1)) // TCBLK
    s = jnp.stack([nblk, k_total])

    W2 = jnp.concatenate([Wv, Wa], axis=1)
    b2 = jnp.concatenate([bv, ba]).reshape(1, 1 + A)
    b1r = b1.reshape(1, H)
    lsr = log_std.reshape(1, A)

    valg, logpg, ent = pl.pallas_call(
        _tc_compact,
        grid_spec=pltpu.PrefetchScalarGridSpec(
            num_scalar_prefetch=1,
            grid=(N // TCBLK,),
            in_specs=[
                pl.BlockSpec((TCBLK, D), _clamped),
                pl.BlockSpec((TCBLK, 2 * A), _clamped),
                pl.BlockSpec((D, H), lambda i, s: (0, 0)),
                pl.BlockSpec((1, H), lambda i, s: (0, 0)),
                pl.BlockSpec((H, 1 + A), lambda i, s: (0, 0)),
                pl.BlockSpec((1, 1 + A), lambda i, s: (0, 0)),
                pl.BlockSpec((1, A), lambda i, s: (0, 0)),
            ],
            out_specs=[
                pl.BlockSpec((TCBLK, 1), lambda i, s: (i, 0)),
                pl.BlockSpec((TCBLK, 1), lambda i, s: (i, 0)),
                pl.BlockSpec((1, 1), lambda i, s: (0, 0)),
            ],
        ),
        out_shape=[
            jax.ShapeDtypeStruct((N, 1), jnp.float32),
            jax.ShapeDtypeStruct((N, 1), jnp.float32),
            jax.ShapeDtypeStruct((1, 1), jnp.float32),
        ],
    )(s, xg, actg, W1, b1r, W2, b2, lsr)

    val1, logp1 = _sc_scatter(perm, valg.reshape(N), logpg.reshape(N), kout)
    return (val1.reshape(N, 1), logp1.reshape(N, 1), ent[0, 0])


# speedup vs baseline: 1.8541x; 1.8541x over previous
import functools

import jax
import jax.numpy as jnp
import numpy as np
from jax import lax
from jax.experimental import pallas as pl
from jax.experimental.pallas import tpu as pltpu
from jax.experimental.pallas import tpu_sc as plsc

N = 8192
D = 1024
H = 2048
A = 64

NC = 2
NS = 16
NW = NC * NS
L = 16
RPT = N // NW
JPT = RPT // L
TCBLK = 256
CH = 32
NCHUNK = RPT // CH

_LOG2PI = float(np.log(2.0 * np.pi))

_MESH = plsc.VectorSubcoreMesh(core_axis_name="c", subcore_axis_name="s")


def _wid():
    return lax.axis_index("s") * NC + lax.axis_index("c")


def _mask01(v):
    return jnp.minimum(jnp.maximum(v, 0), 1)


def _scan16(x, buf2):
    y = x
    for d in (1, 2, 4, 8):
        buf2[pl.ds(L, L)] = y
        y = y + buf2[pl.ds(L - d, L)]
    return y


@functools.partial(
    pl.kernel, mesh=_MESH,
    out_type=jax.ShapeDtypeStruct((NW * L,), jnp.int32),
    scratch_types=[pltpu.VMEM((RPT,), jnp.int32), pltpu.VMEM((L,), jnp.int32)],
)
def _sc_count(stage_hbm, counts_hbm, svm, cbuf):
    wid = _wid()
    pltpu.sync_copy(stage_hbm.at[pl.ds(wid * RPT, RPT)], svm)
    k = jnp.zeros((L,), jnp.int32)
    for j in range(JPT):
        k = k + _mask01(svm[pl.ds(j * L, L)])
    cbuf[...] = k
    pltpu.sync_copy(cbuf, counts_hbm.at[pl.ds(wid * L, L)])


@functools.partial(
    pl.kernel, mesh=_MESH,
    out_type=[jax.ShapeDtypeStruct((N,), jnp.int32),
              jax.ShapeDtypeStruct((L,), jnp.int32)],
    scratch_types=[
        pltpu.VMEM((RPT,), jnp.int32),
        pltpu.VMEM((NW * L,), jnp.int32),
        pltpu.VMEM((NW * L,), jnp.int32),
        pltpu.VMEM((2 * L,), jnp.int32),
        pltpu.VMEM((RPT,), jnp.int32),
        pltpu.VMEM((RPT,), jnp.int32),
        pltpu.VMEM((L,), jnp.int32),
        pltpu.VMEM_SHARED((N,), jnp.int32),
        pltpu.SemaphoreType.DMA,
    ],
)
def _sc_partition(stage_hbm, counts_hbm, perm_hbm, kout_hbm,
                  svm, cvm, offs_vm, buf2, rbuf, pbuf, kbuf, sperm, sem):
    sid = lax.axis_index("s")
    cid = lax.axis_index("c")
    pltpu.sync_copy(counts_hbm, cvm)
    buf2[pl.ds(0, L)] = jnp.zeros((L,), jnp.int32)

    run = jnp.int32(0)
    for b in range(NW):
        v = cvm[pl.ds(b * L, L)]
        s = _scan16(v, buf2)
        offs_vm[pl.ds(b * L, L)] = s - v + run
        buf2[pl.ds(L, L)] = s
        s_ld = buf2[pl.ds(L, L)]
        run = run + s_ld[L - 1]
    k_total = run

    iota = lax.iota(jnp.int32, L)
    for w in (2 * sid, 2 * sid + 1):
        pltpu.sync_copy(stage_hbm.at[pl.ds(w * RPT, RPT)], svm)
        ex = offs_vm[pl.ds(w * L, L)]
        lane_idx = (w * L + iota) * JPT
        iex = k_total + lane_idx - ex
        arun = jnp.zeros((L,), jnp.int32)
        for j in range(JPT):
            v = svm[pl.ds(j * L, L)]
            mi = _mask01(v)
            pos_i = iex + (j - arun)
            pos_a = ex + arun
            pos = pos_i + mi * (pos_a - pos_i)
            rbuf[pl.ds(j * L, L)] = (w * RPT + j * L) + iota
            pbuf[pl.ds(j * L, L)] = pos
            arun = arun + mi
        pltpu.sync_copy(rbuf, sperm.at[pbuf])
    plsc.subcore_barrier()
    half = cid * (N // 2) + sid * RPT
    pltpu.sync_copy(sperm.at[pl.ds(half, RPT)], rbuf)
    pltpu.sync_copy(rbuf, perm_hbm.at[pl.ds(half, RPT)])

    @pl.when(sid + cid == 0)
    def _():
        kbuf[...] = jnp.zeros((L,), jnp.int32) + k_total
        pltpu.sync_copy(kbuf, kout_hbm)


@functools.partial(
    pl.kernel, mesh=_MESH,
    out_type=[jax.ShapeDtypeStruct((N, D), jnp.float32),
              jax.ShapeDtypeStruct((N, 2 * A), jnp.float32)],
    scratch_types=[
        pltpu.VMEM((RPT,), jnp.int32),
        pltpu.VMEM((CH, D), jnp.float32),
        pltpu.VMEM((CH, D), jnp.float32),
        pltpu.VMEM((CH, 2 * A), jnp.float32),
        pltpu.VMEM((CH, 2 * A), jnp.float32),
        pltpu.VMEM((L,), jnp.int32),
        pltpu.SemaphoreType.DMA,
        pltpu.SemaphoreType.DMA,
    ],
)
def _sc_gather(x_hbm, act_hbm, perm_hbm, kq_hbm, xg_hbm, actg_hbm,
               idxv, xb0, xb1, ab0, ab1, kvm, sem0, sem1):
    wid = _wid()
    pltpu.sync_copy(kq_hbm, kvm)
    kv = kvm[...]
    k_total = kv[0]
    nblk = (k_total + (TCBLK - 1)) // TCBLK
    rows_needed = nblk * TCBLK

    @pl.when(wid * RPT < rows_needed)
    def _():
        pltpu.sync_copy(perm_hbm.at[pl.ds(wid * RPT, RPT)], idxv)
        wb = []
        for c in range(NCHUNK):
            xb = xb0 if c % 2 == 0 else xb1
            ab = ab0 if c % 2 == 0 else ab1
            sem = sem0 if c % 2 == 0 else sem1
            if c >= 2:
                wb[2 * (c - 2)].wait()
                wb[2 * (c - 2) + 1].wait()
            idx_c = idxv.at[pl.ds(c * CH, CH)]
            hx = pltpu.async_copy(x_hbm.at[idx_c], xb, sem)
            ha = pltpu.async_copy(act_hbm.at[idx_c], ab, sem)
            hx.wait()
            ha.wait()
            row0 = wid * RPT + c * CH
            wb.append(pltpu.async_copy(xb, xg_hbm.at[pl.ds(row0, CH)], sem))
            wb.append(pltpu.async_copy(ab, actg_hbm.at[pl.ds(row0, CH)], sem))
        for hdl in wb[-4:]:
            hdl.wait()


@functools.partial(
    pl.kernel, mesh=_MESH,
    out_type=[jax.ShapeDtypeStruct((N,), jnp.float32),
              jax.ShapeDtypeStruct((N,), jnp.float32)],
    scratch_types=[
        pltpu.VMEM((RPT,), jnp.int32),
        pltpu.VMEM((RPT,), jnp.float32),
        pltpu.VMEM((RPT,), jnp.float32),
        pltpu.VMEM((RPT,), jnp.float32),
        pltpu.VMEM((RPT,), jnp.float32),
        pltpu.VMEM((L,), jnp.int32),
        pltpu.VMEM_SHARED((N,), jnp.float32),
        pltpu.VMEM_SHARED((N,), jnp.float32),
    ],
)
def _sc_scatter(perm_hbm, valg_hbm, logpg_hbm, kq_hbm, val_hbm, logp_hbm,
                pvm, vvm, lvm, vbuf, lbuf, kvm, sval, slogp):
    sid = lax.axis_index("s")
    cid = lax.axis_index("c")
    pltpu.sync_copy(kq_hbm, kvm)
    k_vec = kvm[...]
    iota = lax.iota(jnp.int32, L)
    for w in (2 * sid, 2 * sid + 1):
        pltpu.sync_copy(perm_hbm.at[pl.ds(w * RPT, RPT)], pvm)
        pltpu.sync_copy(valg_hbm.at[pl.ds(w * RPT, RPT)], vvm)
        pltpu.sync_copy(logpg_hbm.at[pl.ds(w * RPT, RPT)], lvm)
        for j in range(JPT):
            gidx = (w * RPT + j * L) + iota
            live = _mask01(k_vec - gidx).astype(jnp.float32)
            vbuf[pl.ds(j * L, L)] = vvm[pl.ds(j * L, L)] * live
            lbuf[pl.ds(j * L, L)] = lvm[pl.ds(j * L, L)] * live
        pltpu.sync_copy(vbuf, sval.at[pvm])
        pltpu.sync_copy(lbuf, slogp.at[pvm])
    plsc.subcore_barrier()
    half = cid * (N // 2) + sid * RPT
    pltpu.sync_copy(sval.at[pl.ds(half, RPT)], vbuf)
    pltpu.sync_copy(vbuf, val_hbm.at[pl.ds(half, RPT)])
    pltpu.sync_copy(slogp.at[pl.ds(half, RPT)], lbuf)
    pltpu.sync_copy(lbuf, logp_hbm.at[pl.ds(half, RPT)])


def _tc_compact(s_ref, xg_ref, actg_ref, w1_ref, b1_ref, w2_ref, b2_ref,
                logstd_ref, valg_ref, logpg_ref, ent_ref):
    i = pl.program_id(0)
    log_std = logstd_ref[...]
    sum_log_std = jnp.sum(log_std)

    @pl.when(i < s_ref[0])
    def _():
        h = jnp.tanh(jnp.dot(xg_ref[...], w1_ref[...],
                             preferred_element_type=jnp.float32) + b1_ref[...])
        out2 = (jnp.dot(h, w2_ref[...], preferred_element_type=jnp.float32)
                + b2_ref[...])
        val = out2[:, 0:1]
        mu = out2[:, 1:1 + A]
        inv_std = jnp.exp(-log_std)
        diff = (actg_ref[:, 0:A] - mu) * inv_std
        valg_ref[...] = val
        logpg_ref[...] = (-0.5 * jnp.sum(diff * diff, axis=-1, keepdims=True)
                          - sum_log_std - 0.5 * A * _LOG2PI)

    @pl.when(i >= s_ref[0])
    def _():
        valg_ref[...] = jnp.zeros((TCBLK, 1), jnp.float32)
        logpg_ref[...] = jnp.zeros((TCBLK, 1), jnp.float32)

    @pl.when(i == 0)
    def _():
        ent_scalar = sum_log_std + 0.5 * A * (_LOG2PI + 1.0)
        ent_ref[...] = (s_ref[1].astype(jnp.float32)
                        * (ent_scalar / N)).reshape(1, 1)


def _clamped(i, s):
    return (jnp.maximum(jnp.minimum(i, s[0] - 1), 0), 0)


def kernel(stage, x, act, W1, b1, Wv, bv, Wa, ba, log_std):
    stage_i = stage.astype(jnp.int32)
    counts = _sc_count(stage_i)
    perm, kout = _sc_partition(stage_i, counts)
    act128 = jnp.pad(act, ((0, 0), (0, A)))
    xg, actg = _sc_gather(x, act128, perm, kout)

    k_total = kout[0]
    nblk = (k_total + (TCBLK - 1)) // TCBLK
    s = jnp.stack([nblk, k_total])

    W2 = jnp.concatenate([Wv, Wa], axis=1)
    b2 = jnp.concatenate([bv, ba]).reshape(1, 1 + A)
    b1r = b1.reshape(1, H)
    lsr = log_std.reshape(1, A)

    valg, logpg, ent = pl.pallas_call(
        _tc_compact,
        grid_spec=pltpu.PrefetchScalarGridSpec(
            num_scalar_prefetch=1,
            grid=(N // TCBLK,),
            in_specs=[
                pl.BlockSpec((TCBLK, D), _clamped),
                pl.BlockSpec((TCBLK, 2 * A), _clamped),
                pl.BlockSpec((D, H), lambda i, s: (0, 0)),
                pl.BlockSpec((1, H), lambda i, s: (0, 0)),
                pl.BlockSpec((H, 1 + A), lambda i, s: (0, 0)),
                pl.BlockSpec((1, 1 + A), lambda i, s: (0, 0)),
                pl.BlockSpec((1, A), lambda i, s: (0, 0)),
            ],
            out_specs=[
                pl.BlockSpec((TCBLK, 1), lambda i, s: (i, 0)),
                pl.BlockSpec((TCBLK, 1), lambda i, s: (i, 0)),
                pl.BlockSpec((1, 1), lambda i, s: (0, 0)),
            ],
        ),
        out_shape=[
            jax.ShapeDtypeStruct((N, 1), jnp.float32),
            jax.ShapeDtypeStruct((N, 1), jnp.float32),
            jax.ShapeDtypeStruct((1, 1), jnp.float32),
        ],
    )(s, xg, actg, W1, b1r, W2, b2, lsr)

    val1, logp1 = _sc_scatter(perm, valg.reshape(N), logpg.reshape(N), kout)
    return (val1.reshape(N, 1), logp1.reshape(N, 1), ent[0, 0])


# speedup vs baseline: 1.9195x; 1.0353x over previous
import functools

import jax
import jax.numpy as jnp
import numpy as np
from jax import lax
from jax.experimental import pallas as pl
from jax.experimental.pallas import tpu as pltpu
from jax.experimental.pallas import tpu_sc as plsc

N = 8192
D = 1024
H = 2048
A = 64

NC = 2
NS = 16
NW = NC * NS
L = 16
RPT = N // NW
JPT = RPT // L
TCBLK = 256
CH = 32
NCHUNK = RPT // CH

_LOG2PI = float(np.log(2.0 * np.pi))

_MESH = plsc.VectorSubcoreMesh(core_axis_name="c", subcore_axis_name="s")


def _wid():
    return lax.axis_index("s") * NC + lax.axis_index("c")


def _mask01(v):
    return jnp.minimum(jnp.maximum(v, 0), 1)


def _scan16(x, buf2):
    y = x
    for d in (1, 2, 4, 8):
        buf2[pl.ds(L, L)] = y
        y = y + buf2[pl.ds(L - d, L)]
    return y


@functools.partial(
    pl.kernel, mesh=_MESH,
    out_type=jax.ShapeDtypeStruct((NW * L,), jnp.int32),
    scratch_types=[pltpu.VMEM((RPT,), jnp.int32), pltpu.VMEM((L,), jnp.int32)],
)
def _sc_count(stage_hbm, counts_hbm, svm, cbuf):
    wid = _wid()
    pltpu.sync_copy(stage_hbm.at[pl.ds(wid * RPT, RPT)], svm)
    k = jnp.zeros((L,), jnp.int32)
    for j in range(JPT):
        k = k + _mask01(svm[pl.ds(j * L, L)])
    cbuf[...] = k
    pltpu.sync_copy(cbuf, counts_hbm.at[pl.ds(wid * L, L)])


@functools.partial(
    pl.kernel, mesh=_MESH,
    out_type=[jax.ShapeDtypeStruct((N,), jnp.int32),
              jax.ShapeDtypeStruct((L,), jnp.int32)],
    scratch_types=[
        pltpu.VMEM((RPT,), jnp.int32),
        pltpu.VMEM((NW * L,), jnp.int32),
        pltpu.VMEM((NW * L,), jnp.int32),
        pltpu.VMEM((2 * L,), jnp.int32),
        pltpu.VMEM((RPT,), jnp.int32),
        pltpu.VMEM((RPT,), jnp.int32),
        pltpu.VMEM((L,), jnp.int32),
        pltpu.VMEM_SHARED((N,), jnp.int32),
        pltpu.SemaphoreType.DMA,
    ],
)
def _sc_partition(stage_hbm, counts_hbm, perm_hbm, kout_hbm,
                  svm, cvm, offs_vm, buf2, rbuf, pbuf, kbuf, sperm, sem):
    sid = lax.axis_index("s")
    cid = lax.axis_index("c")
    pltpu.sync_copy(counts_hbm, cvm)
    buf2[pl.ds(0, L)] = jnp.zeros((L,), jnp.int32)

    run = jnp.int32(0)
    for b in range(NW):
        v = cvm[pl.ds(b * L, L)]
        s = _scan16(v, buf2)
        offs_vm[pl.ds(b * L, L)] = s - v + run
        buf2[pl.ds(L, L)] = s
        s_ld = buf2[pl.ds(L, L)]
        run = run + s_ld[L - 1]
    k_total = run

    iota = lax.iota(jnp.int32, L)
    for w in (2 * sid, 2 * sid + 1):
        pltpu.sync_copy(stage_hbm.at[pl.ds(w * RPT, RPT)], svm)
        ex = offs_vm[pl.ds(w * L, L)]
        lane_idx = (w * L + iota) * JPT
        iex = k_total + lane_idx - ex
        arun = jnp.zeros((L,), jnp.int32)
        for j in range(JPT):
            v = svm[pl.ds(j * L, L)]
            mi = _mask01(v)
            pos_i = iex + (j - arun)
            pos_a = ex + arun
            pos = pos_i + mi * (pos_a - pos_i)
            rbuf[pl.ds(j * L, L)] = (w * RPT + j * L) + iota
            pbuf[pl.ds(j * L, L)] = pos
            arun = arun + mi
        pltpu.sync_copy(rbuf, sperm.at[pbuf])
    plsc.subcore_barrier()
    half = cid * (N // 2) + sid * RPT
    pltpu.sync_copy(sperm.at[pl.ds(half, RPT)], rbuf)
    pltpu.sync_copy(rbuf, perm_hbm.at[pl.ds(half, RPT)])

    @pl.when(sid + cid == 0)
    def _():
        kbuf[...] = jnp.zeros((L,), jnp.int32) + k_total
        pltpu.sync_copy(kbuf, kout_hbm)


@functools.partial(
    pl.kernel, mesh=_MESH,
    out_type=[jax.ShapeDtypeStruct((N, D), jnp.float32),
              jax.ShapeDtypeStruct((N, 2 * A), jnp.float32)],
    scratch_types=[
        pltpu.VMEM((RPT,), jnp.int32),
        pltpu.VMEM((CH, D), jnp.float32),
        pltpu.VMEM((CH, D), jnp.float32),
        pltpu.VMEM((CH, 2 * A), jnp.float32),
        pltpu.VMEM((CH, 2 * A), jnp.float32),
        pltpu.VMEM((L,), jnp.int32),
        pltpu.SemaphoreType.DMA,
        pltpu.SemaphoreType.DMA,
    ],
)
def _sc_gather(x_hbm, act_hbm, perm_hbm, kq_hbm, xg_hbm, actg_hbm,
               idxv, xb0, xb1, ab0, ab1, kvm, sem0, sem1):
    wid = _wid()
    pltpu.sync_copy(kq_hbm, kvm)
    kv = kvm[...]
    k_total = kv[0]
    nblk = (k_total + (TCBLK - 1)) // TCBLK
    rows_needed = nblk * TCBLK

    @pl.when(wid * RPT < rows_needed)
    def _():
        pltpu.sync_copy(perm_hbm.at[pl.ds(wid * RPT, RPT)], idxv)
        wb = []
        for c in range(NCHUNK):
            xb = xb0 if c % 2 == 0 else xb1
            ab = ab0 if c % 2 == 0 else ab1
            sem = sem0 if c % 2 == 0 else sem1
            if c >= 2:
                wb[2 * (c - 2)].wait()
                wb[2 * (c - 2) + 1].wait()
            idx_c = idxv.at[pl.ds(c * CH, CH)]
            hx = pltpu.async_copy(x_hbm.at[idx_c], xb, sem)
            ha = pltpu.async_copy(act_hbm.at[idx_c], ab, sem)
            hx.wait()
            ha.wait()
            row0 = wid * RPT + c * CH
            wb.append(pltpu.async_copy(xb, xg_hbm.at[pl.ds(row0, CH)], sem))
            wb.append(pltpu.async_copy(ab, actg_hbm.at[pl.ds(row0, CH)], sem))
        for hdl in wb[-4:]:
            hdl.wait()


@functools.partial(
    pl.kernel, mesh=_MESH,
    out_type=[jax.ShapeDtypeStruct((N,), jnp.float32),
              jax.ShapeDtypeStruct((N,), jnp.float32)],
    scratch_types=[
        pltpu.VMEM((RPT,), jnp.int32),
        pltpu.VMEM((RPT,), jnp.float32),
        pltpu.VMEM((RPT,), jnp.float32),
        pltpu.VMEM((RPT,), jnp.float32),
        pltpu.VMEM((RPT,), jnp.float32),
        pltpu.VMEM((L,), jnp.int32),
        pltpu.VMEM_SHARED((N,), jnp.float32),
        pltpu.VMEM_SHARED((N,), jnp.float32),
    ],
)
def _sc_scatter(perm_hbm, valg_hbm, logpg_hbm, kq_hbm, val_hbm, logp_hbm,
                pvm, vvm, lvm, vbuf, lbuf, kvm, sval, slogp):
    sid = lax.axis_index("s")
    cid = lax.axis_index("c")
    pltpu.sync_copy(kq_hbm, kvm)
    k_vec = kvm[...]
    iota = lax.iota(jnp.int32, L)
    for w in (2 * sid, 2 * sid + 1):
        pltpu.sync_copy(perm_hbm.at[pl.ds(w * RPT, RPT)], pvm)
        pltpu.sync_copy(valg_hbm.at[pl.ds(w * RPT, RPT)], vvm)
        pltpu.sync_copy(logpg_hbm.at[pl.ds(w * RPT, RPT)], lvm)
        for j in range(JPT):
            gidx = (w * RPT + j * L) + iota
            live = _mask01(k_vec - gidx).astype(jnp.float32)
            vbuf[pl.ds(j * L, L)] = vvm[pl.ds(j * L, L)] * live
            lbuf[pl.ds(j * L, L)] = lvm[pl.ds(j * L, L)] * live
        pltpu.sync_copy(vbuf, sval.at[pvm])
        pltpu.sync_copy(lbuf, slogp.at[pvm])
    plsc.subcore_barrier()
    half = cid * (N // 2) + sid * RPT
    pltpu.sync_copy(sval.at[pl.ds(half, RPT)], vbuf)
    pltpu.sync_copy(vbuf, val_hbm.at[pl.ds(half, RPT)])
    pltpu.sync_copy(slogp.at[pl.ds(half, RPT)], lbuf)
    pltpu.sync_copy(lbuf, logp_hbm.at[pl.ds(half, RPT)])


def _tc_compact(s_ref, xg_ref, actg_ref, w1_ref, b1_ref, w2_ref, b2_ref,
                logstd_ref, valg_ref, logpg_ref, ent_ref):
    i = pl.program_id(0)
    log_std = logstd_ref[...]
    sum_log_std = jnp.sum(log_std)

    @pl.when(i < s_ref[0])
    def _():
        h = jnp.tanh(jnp.dot(xg_ref[...], w1_ref[...],
                             preferred_element_type=jnp.float32) + b1_ref[...])
        out2 = (jnp.dot(h, w2_ref[...], preferred_element_type=jnp.float32)
                + b2_ref[...])
        val = out2[:, 0:1]
        mu = out2[:, 1:1 + A]
        inv_std = jnp.exp(-log_std)
        diff = (actg_ref[:, 0:A] - mu) * inv_std
        valg_ref[...] = val.reshape(TCBLK)
        logpg_ref[...] = (-0.5 * jnp.sum(diff * diff, axis=-1)
                          - sum_log_std - 0.5 * A * _LOG2PI)

    @pl.when(i >= s_ref[0])
    def _():
        valg_ref[...] = jnp.zeros((TCBLK,), jnp.float32)
        logpg_ref[...] = jnp.zeros((TCBLK,), jnp.float32)

    @pl.when(i == 0)
    def _():
        ent_scalar = sum_log_std + 0.5 * A * (_LOG2PI + 1.0)
        ent_ref[...] = (s_ref[1].astype(jnp.float32)
                        * (ent_scalar / N)).reshape(1, 1)


def _clamped(i, s):
    return (jnp.maximum(jnp.minimum(i, s[0] - 1), 0), 0)


def kernel(stage, x, act, W1, b1, Wv, bv, Wa, ba, log_std):
    stage_i = stage.astype(jnp.int32)
    counts = _sc_count(stage_i)
    perm, kout = _sc_partition(stage_i, counts)
    act128 = jnp.pad(act, ((0, 0), (0, A)))
    xg, actg = _sc_gather(x, act128, perm, kout)

    k_total = kout[0]
    nblk = (k_total + (TCBLK - 1)) // TCBLK
    s = jnp.stack([nblk, k_total])

    W2 = jnp.concatenate([Wv, Wa], axis=1)
    b2 = jnp.concatenate([bv, ba]).reshape(1, 1 + A)
    b1r = b1.reshape(1, H)
    lsr = log_std.reshape(1, A)

    valg, logpg, ent = pl.pallas_call(
        _tc_compact,
        grid_spec=pltpu.PrefetchScalarGridSpec(
            num_scalar_prefetch=1,
            grid=(N // TCBLK,),
            in_specs=[
                pl.BlockSpec((TCBLK, D), _clamped),
                pl.BlockSpec((TCBLK, 2 * A), _clamped),
                pl.BlockSpec((D, H), lambda i, s: (0, 0)),
                pl.BlockSpec((1, H), lambda i, s: (0, 0)),
                pl.BlockSpec((H, 1 + A), lambda i, s: (0, 0)),
                pl.BlockSpec((1, 1 + A), lambda i, s: (0, 0)),
                pl.BlockSpec((1, A), lambda i, s: (0, 0)),
            ],
            out_specs=[
                pl.BlockSpec((TCBLK,), lambda i, s: (i,)),
                pl.BlockSpec((TCBLK,), lambda i, s: (i,)),
                pl.BlockSpec((1, 1), lambda i, s: (0, 0)),
            ],
        ),
        out_shape=[
            jax.ShapeDtypeStruct((N,), jnp.float32),
            jax.ShapeDtypeStruct((N,), jnp.float32),
            jax.ShapeDtypeStruct((1, 1), jnp.float32),
        ],
    )(s, xg, actg, W1, b1r, W2, b2, lsr)

    val1, logp1 = _sc_scatter(perm, valg, logpg, kout)
    return (val1.reshape(N, 1), logp1.reshape(N, 1), ent[0, 0])


# speedup vs baseline: 1.9608x; 1.0215x over previous
import functools

import jax
import jax.numpy as jnp
import numpy as np
from jax import lax
from jax.experimental import pallas as pl
from jax.experimental.pallas import tpu as pltpu
from jax.experimental.pallas import tpu_sc as plsc

N = 8192
D = 1024
H = 2048
A = 64

NC = 2
NS = 16
NW = NC * NS
L = 16
RPT = N // NW
JPT = RPT // L
TCBLK = 256
CH = 32
NCHUNK = RPT // CH

_LOG2PI = float(np.log(2.0 * np.pi))

_MESH = plsc.VectorSubcoreMesh(core_axis_name="c", subcore_axis_name="s")


def _wid():
    return lax.axis_index("s") * NC + lax.axis_index("c")


def _mask01(v):
    return jnp.minimum(jnp.maximum(v, 0), 1)


def _scan16(x, buf2):
    y = x
    for d in (1, 2, 4, 8):
        buf2[pl.ds(L, L)] = y
        y = y + buf2[pl.ds(L - d, L)]
    return y


@functools.partial(
    pl.kernel, mesh=_MESH,
    out_type=jax.ShapeDtypeStruct((NW * L,), jnp.int32),
    scratch_types=[pltpu.VMEM((RPT,), jnp.int32), pltpu.VMEM((L,), jnp.int32)],
)
def _sc_count(stage_hbm, counts_hbm, svm, cbuf):
    wid = _wid()
    pltpu.sync_copy(stage_hbm.at[pl.ds(wid * RPT, RPT)], svm)
    k = jnp.zeros((L,), jnp.int32)
    for j in range(JPT):
        k = k + _mask01(svm[pl.ds(j * L, L)])
    cbuf[...] = k
    pltpu.sync_copy(cbuf, counts_hbm.at[pl.ds(wid * L, L)])


@functools.partial(
    pl.kernel, mesh=_MESH,
    out_type=[jax.ShapeDtypeStruct((N,), jnp.int32),
              jax.ShapeDtypeStruct((L,), jnp.int32)],
    scratch_types=[
        pltpu.VMEM((RPT,), jnp.int32),
        pltpu.VMEM((NW * L,), jnp.int32),
        pltpu.VMEM((NW * L,), jnp.int32),
        pltpu.VMEM((2 * L,), jnp.int32),
        pltpu.VMEM((RPT,), jnp.int32),
        pltpu.VMEM((RPT,), jnp.int32),
        pltpu.VMEM((L,), jnp.int32),
        pltpu.VMEM_SHARED((N,), jnp.int32),
        pltpu.SemaphoreType.DMA,
    ],
)
def _sc_partition(stage_hbm, counts_hbm, perm_hbm, kout_hbm,
                  svm, cvm, offs_vm, buf2, rbuf, pbuf, kbuf, sperm, sem):
    sid = lax.axis_index("s")
    cid = lax.axis_index("c")
    pltpu.sync_copy(counts_hbm, cvm)
    buf2[pl.ds(0, L)] = jnp.zeros((L,), jnp.int32)

    run = jnp.int32(0)
    for b in range(NW):
        v = cvm[pl.ds(b * L, L)]
        s = _scan16(v, buf2)
        offs_vm[pl.ds(b * L, L)] = s - v + run
        buf2[pl.ds(L, L)] = s
        s_ld = buf2[pl.ds(L, L)]
        run = run + s_ld[L - 1]
    k_total = run

    iota = lax.iota(jnp.int32, L)
    for w in (2 * sid, 2 * sid + 1):
        pltpu.sync_copy(stage_hbm.at[pl.ds(w * RPT, RPT)], svm)
        ex = offs_vm[pl.ds(w * L, L)]
        lane_idx = (w * L + iota) * JPT
        iex = k_total + lane_idx - ex
        arun = jnp.zeros((L,), jnp.int32)
        for j in range(JPT):
            v = svm[pl.ds(j * L, L)]
            mi = _mask01(v)
            pos_i = iex + (j - arun)
            pos_a = ex + arun
            pos = pos_i + mi * (pos_a - pos_i)
            rbuf[pl.ds(j * L, L)] = (w * RPT + j * L) + iota
            pbuf[pl.ds(j * L, L)] = pos
            arun = arun + mi
        pltpu.sync_copy(rbuf, sperm.at[pbuf])
    plsc.subcore_barrier()
    half = cid * (N // 2) + sid * RPT
    pltpu.sync_copy(sperm.at[pl.ds(half, RPT)], rbuf)
    pltpu.sync_copy(rbuf, perm_hbm.at[pl.ds(half, RPT)])

    @pl.when(sid + cid == 0)
    def _():
        kbuf[...] = jnp.zeros((L,), jnp.int32) + k_total
        pltpu.sync_copy(kbuf, kout_hbm)


NH = N // 2
RPH = NH // NW
NCH_H = RPH // CH


def _make_gather(base):
    @functools.partial(
        pl.kernel, mesh=_MESH,
        out_type=[jax.ShapeDtypeStruct((NH, D), jnp.float32),
                  jax.ShapeDtypeStruct((NH, 2 * A), jnp.float32)],
        scratch_types=[
            pltpu.VMEM((RPH,), jnp.int32),
            pltpu.VMEM((CH, D), jnp.float32),
            pltpu.VMEM((CH, D), jnp.float32),
            pltpu.VMEM((CH, 2 * A), jnp.float32),
            pltpu.VMEM((CH, 2 * A), jnp.float32),
            pltpu.VMEM((L,), jnp.int32),
            pltpu.SemaphoreType.DMA,
            pltpu.SemaphoreType.DMA,
        ],
    )
    def _g(x_hbm, act_hbm, perm_hbm, kq_hbm, xg_hbm, actg_hbm,
           idxv, xb0, xb1, ab0, ab1, kvm, sem0, sem1):
        wid = _wid()
        pltpu.sync_copy(kq_hbm, kvm)
        kv = kvm[...]
        k_total = kv[0]
        nblk = (k_total + (TCBLK - 1)) // TCBLK
        rows_needed = nblk * TCBLK

        @pl.when(base + wid * RPH < rows_needed)
        def _():
            pltpu.sync_copy(perm_hbm.at[pl.ds(base + wid * RPH, RPH)], idxv)
            wb = []
            for c in range(NCH_H):
                xb = xb0 if c % 2 == 0 else xb1
                ab = ab0 if c % 2 == 0 else ab1
                sem = sem0 if c % 2 == 0 else sem1
                if c >= 2:
                    wb[2 * (c - 2)].wait()
                    wb[2 * (c - 2) + 1].wait()
                idx_c = idxv.at[pl.ds(c * CH, CH)]
                hx = pltpu.async_copy(x_hbm.at[idx_c], xb, sem)
                ha = pltpu.async_copy(act_hbm.at[idx_c], ab, sem)
                hx.wait()
                ha.wait()
                row0 = wid * RPH + c * CH
                wb.append(pltpu.async_copy(xb, xg_hbm.at[pl.ds(row0, CH)], sem))
                wb.append(
                    pltpu.async_copy(ab, actg_hbm.at[pl.ds(row0, CH)], sem))
            for hdl in wb[-4:]:
                hdl.wait()

    return _g


_sc_gather_a = _make_gather(0)
_sc_gather_b = _make_gather(NH)


@functools.partial(
    pl.kernel, mesh=_MESH,
    out_type=[jax.ShapeDtypeStruct((N,), jnp.float32),
              jax.ShapeDtypeStruct((N,), jnp.float32)],
    scratch_types=[
        pltpu.VMEM((RPT,), jnp.int32),
        pltpu.VMEM((RPT,), jnp.float32),
        pltpu.VMEM((RPT,), jnp.float32),
        pltpu.VMEM((RPT,), jnp.float32),
        pltpu.VMEM((RPT,), jnp.float32),
        pltpu.VMEM((L,), jnp.int32),
        pltpu.VMEM_SHARED((N,), jnp.float32),
        pltpu.VMEM_SHARED((N,), jnp.float32),
    ],
)
def _sc_scatter(perm_hbm, valga_hbm, valgb_hbm, logpga_hbm, logpgb_hbm,
                kq_hbm, val_hbm, logp_hbm,
                pvm, vvm, lvm, vbuf, lbuf, kvm, sval, slogp):
    sid = lax.axis_index("s")
    cid = lax.axis_index("c")
    pltpu.sync_copy(kq_hbm, kvm)
    k_vec = kvm[...]
    iota = lax.iota(jnp.int32, L)
    for w in (2 * sid, 2 * sid + 1):
        pltpu.sync_copy(perm_hbm.at[pl.ds(w * RPT, RPT)], pvm)

        @pl.when(sid < NS // 2)
        def _():
            pltpu.sync_copy(valga_hbm.at[pl.ds(w * RPT, RPT)], vvm)
            pltpu.sync_copy(logpga_hbm.at[pl.ds(w * RPT, RPT)], lvm)

        @pl.when(sid >= NS // 2)
        def _():
            pltpu.sync_copy(valgb_hbm.at[pl.ds(w * RPT - NH, RPT)], vvm)
            pltpu.sync_copy(logpgb_hbm.at[pl.ds(w * RPT - NH, RPT)], lvm)
        for j in range(JPT):
            gidx = (w * RPT + j * L) + iota
            live = _mask01(k_vec - gidx).astype(jnp.float32)
            vbuf[pl.ds(j * L, L)] = vvm[pl.ds(j * L, L)] * live
            lbuf[pl.ds(j * L, L)] = lvm[pl.ds(j * L, L)] * live
        pltpu.sync_copy(vbuf, sval.at[pvm])
        pltpu.sync_copy(lbuf, slogp.at[pvm])
    plsc.subcore_barrier()
    half = cid * (N // 2) + sid * RPT
    pltpu.sync_copy(sval.at[pl.ds(half, RPT)], vbuf)
    pltpu.sync_copy(vbuf, val_hbm.at[pl.ds(half, RPT)])
    pltpu.sync_copy(slogp.at[pl.ds(half, RPT)], lbuf)
    pltpu.sync_copy(lbuf, logp_hbm.at[pl.ds(half, RPT)])


def _tc_compact(s_ref, xg_ref, actg_ref, w1_ref, b1_ref, w2_ref, b2_ref,
                logstd_ref, valg_ref, logpg_ref, ent_ref):
    i = pl.program_id(0)
    log_std = logstd_ref[...]
    sum_log_std = jnp.sum(log_std)

    @pl.when(i < s_ref[0])
    def _():
        h = jnp.tanh(jnp.dot(xg_ref[...], w1_ref[...],
                             preferred_element_type=jnp.float32) + b1_ref[...])
        out2 = (jnp.dot(h, w2_ref[...], preferred_element_type=jnp.float32)
                + b2_ref[...])
        val = out2[:, 0:1]
        mu = out2[:, 1:1 + A]
        inv_std = jnp.exp(-log_std)
        diff = (actg_ref[:, 0:A] - mu) * inv_std
        valg_ref[...] = val.reshape(TCBLK)
        logpg_ref[...] = (-0.5 * jnp.sum(diff * diff, axis=-1)
                          - sum_log_std - 0.5 * A * _LOG2PI)

    @pl.when(i >= s_ref[0])
    def _():
        valg_ref[...] = jnp.zeros((TCBLK,), jnp.float32)
        logpg_ref[...] = jnp.zeros((TCBLK,), jnp.float32)

    @pl.when(i == 0)
    def _():
        ent_scalar = sum_log_std + 0.5 * A * (_LOG2PI + 1.0)
        ent_ref[...] = (s_ref[1].astype(jnp.float32)
                        * (ent_scalar / N)).reshape(1, 1)


def _clamped(i, s):
    return (jnp.maximum(jnp.minimum(i, s[0] - 1), 0), 0)


def _tc_half(s, xg, actg, W1, b1r, W2, b2, lsr):
    return pl.pallas_call(
        _tc_compact,
        grid_spec=pltpu.PrefetchScalarGridSpec(
            num_scalar_prefetch=1,
            grid=(NH // TCBLK,),
            in_specs=[
                pl.BlockSpec((TCBLK, D), _clamped),
                pl.BlockSpec((TCBLK, 2 * A), _clamped),
                pl.BlockSpec((D, H), lambda i, s: (0, 0)),
                pl.BlockSpec((1, H), lambda i, s: (0, 0)),
                pl.BlockSpec((H, 1 + A), lambda i, s: (0, 0)),
                pl.BlockSpec((1, 1 + A), lambda i, s: (0, 0)),
                pl.BlockSpec((1, A), lambda i, s: (0, 0)),
            ],
            out_specs=[
                pl.BlockSpec((TCBLK,), lambda i, s: (i,)),
                pl.BlockSpec((TCBLK,), lambda i, s: (i,)),
                pl.BlockSpec((1, 1), lambda i, s: (0, 0)),
            ],
        ),
        out_shape=[
            jax.ShapeDtypeStruct((NH,), jnp.float32),
            jax.ShapeDtypeStruct((NH,), jnp.float32),
            jax.ShapeDtypeStruct((1, 1), jnp.float32),
        ],
    )(s, xg, actg, W1, b1r, W2, b2, lsr)


def kernel(stage, x, act, W1, b1, Wv, bv, Wa, ba, log_std):
    stage_i = stage.astype(jnp.int32)
    counts = _sc_count(stage_i)
    perm, kout = _sc_partition(stage_i, counts)
    act128 = jnp.pad(act, ((0, 0), (0, A)))
    xga, actga = _sc_gather_a(x, act128, perm, kout)
    xgb, actgb = _sc_gather_b(x, act128, perm, kout)

    k_total = kout[0]
    nblk = (k_total + (TCBLK - 1)) // TCBLK
    nbh = NH // TCBLK
    s_a = jnp.stack([jnp.minimum(nblk, nbh), k_total])
    s_b = jnp.stack([jnp.clip(nblk - nbh, 0, nbh), k_total])

    W2 = jnp.concatenate([Wv, Wa], axis=1)
    b2 = jnp.concatenate([bv, ba]).reshape(1, 1 + A)
    b1r = b1.reshape(1, H)
    lsr = log_std.reshape(1, A)

    valga, logpga, ent = _tc_half(s_a, xga, actga, W1, b1r, W2, b2, lsr)
    valgb, logpgb, _ = _tc_half(s_b, xgb, actgb, W1, b1r, W2, b2, lsr)

    val1, logp1 = _sc_scatter(perm, valga, valgb, logpga, logpgb, kout)
    return (val1.reshape(N, 1), logp1.reshape(N, 1), ent[0, 0])


# speedup vs baseline: 2.0021x; 1.0210x over previous
import functools

import jax
import jax.numpy as jnp
import numpy as np
from jax import lax
from jax.experimental import pallas as pl
from jax.experimental.pallas import tpu as pltpu
from jax.experimental.pallas import tpu_sc as plsc

N = 8192
D = 1024
H = 2048
A = 64

NC = 2
NS = 16
NW = NC * NS
L = 16
RPT = N // NW
JPT = RPT // L
TCBLK = 512
CH = 32
NCHUNK = RPT // CH

_LOG2PI = float(np.log(2.0 * np.pi))

_MESH = plsc.VectorSubcoreMesh(core_axis_name="c", subcore_axis_name="s")


def _wid():
    return lax.axis_index("s") * NC + lax.axis_index("c")


def _mask01(v):
    return jnp.minimum(jnp.maximum(v, 0), 1)


def _scan16(x, buf2):
    y = x
    for d in (1, 2, 4, 8):
        buf2[pl.ds(L, L)] = y
        y = y + buf2[pl.ds(L - d, L)]
    return y


@functools.partial(
    pl.kernel, mesh=_MESH,
    out_type=jax.ShapeDtypeStruct((NW * L,), jnp.int32),
    scratch_types=[pltpu.VMEM((RPT,), jnp.int32), pltpu.VMEM((L,), jnp.int32)],
)
def _sc_count(stage_hbm, counts_hbm, svm, cbuf):
    wid = _wid()
    pltpu.sync_copy(stage_hbm.at[pl.ds(wid * RPT, RPT)], svm)
    k = jnp.zeros((L,), jnp.int32)
    for j in range(JPT):
        k = k + _mask01(svm[pl.ds(j * L, L)])
    cbuf[...] = k
    pltpu.sync_copy(cbuf, counts_hbm.at[pl.ds(wid * L, L)])


@functools.partial(
    pl.kernel, mesh=_MESH,
    out_type=[jax.ShapeDtypeStruct((N,), jnp.int32),
              jax.ShapeDtypeStruct((L,), jnp.int32)],
    scratch_types=[
        pltpu.VMEM((RPT,), jnp.int32),
        pltpu.VMEM((NW * L,), jnp.int32),
        pltpu.VMEM((NW * L,), jnp.int32),
        pltpu.VMEM((2 * L,), jnp.int32),
        pltpu.VMEM((RPT,), jnp.int32),
        pltpu.VMEM((RPT,), jnp.int32),
        pltpu.VMEM((L,), jnp.int32),
        pltpu.VMEM_SHARED((N,), jnp.int32),
        pltpu.SemaphoreType.DMA,
    ],
)
def _sc_partition(stage_hbm, counts_hbm, perm_hbm, kout_hbm,
                  svm, cvm, offs_vm, buf2, rbuf, pbuf, kbuf, sperm, sem):
    sid = lax.axis_index("s")
    cid = lax.axis_index("c")
    pltpu.sync_copy(counts_hbm, cvm)
    buf2[pl.ds(0, L)] = jnp.zeros((L,), jnp.int32)

    run = jnp.int32(0)
    for b in range(NW):
        v = cvm[pl.ds(b * L, L)]
        s = _scan16(v, buf2)
        offs_vm[pl.ds(b * L, L)] = s - v + run
        buf2[pl.ds(L, L)] = s
        s_ld = buf2[pl.ds(L, L)]
        run = run + s_ld[L - 1]
    k_total = run

    iota = lax.iota(jnp.int32, L)
    for w in (2 * sid, 2 * sid + 1):
        pltpu.sync_copy(stage_hbm.at[pl.ds(w * RPT, RPT)], svm)
        ex = offs_vm[pl.ds(w * L, L)]
        lane_idx = (w * L + iota) * JPT
        iex = k_total + lane_idx - ex
        arun = jnp.zeros((L,), jnp.int32)
        for j in range(JPT):
            v = svm[pl.ds(j * L, L)]
            mi = _mask01(v)
            pos_i = iex + (j - arun)
            pos_a = ex + arun
            pos = pos_i + mi * (pos_a - pos_i)
            rbuf[pl.ds(j * L, L)] = (w * RPT + j * L) + iota
            pbuf[pl.ds(j * L, L)] = pos
            arun = arun + mi
        pltpu.sync_copy(rbuf, sperm.at[pbuf])
    plsc.subcore_barrier()
    half = cid * (N // 2) + sid * RPT
    pltpu.sync_copy(sperm.at[pl.ds(half, RPT)], rbuf)
    pltpu.sync_copy(rbuf, perm_hbm.at[pl.ds(half, RPT)])

    @pl.when(sid + cid == 0)
    def _():
        kbuf[...] = jnp.zeros((L,), jnp.int32) + k_total
        pltpu.sync_copy(kbuf, kout_hbm)


NH = N // 2
RPH = NH // NW
NCH_H = RPH // CH


def _make_gather(base):
    @functools.partial(
        pl.kernel, mesh=_MESH,
        out_type=[jax.ShapeDtypeStruct((NH, D), jnp.float32),
                  jax.ShapeDtypeStruct((NH, 2 * A), jnp.float32)],
        scratch_types=[
            pltpu.VMEM((RPH,), jnp.int32),
            pltpu.VMEM((CH, D), jnp.float32),
            pltpu.VMEM((CH, D), jnp.float32),
            pltpu.VMEM((CH, 2 * A), jnp.float32),
            pltpu.VMEM((CH, 2 * A), jnp.float32),
            pltpu.VMEM((L,), jnp.int32),
            pltpu.SemaphoreType.DMA,
            pltpu.SemaphoreType.DMA,
        ],
    )
    def _g(x_hbm, act_hbm, perm_hbm, kq_hbm, xg_hbm, actg_hbm,
           idxv, xb0, xb1, ab0, ab1, kvm, sem0, sem1):
        wid = _wid()
        pltpu.sync_copy(kq_hbm, kvm)
        kv = kvm[...]
        k_total = kv[0]
        nblk = (k_total + (TCBLK - 1)) // TCBLK
        rows_needed = nblk * TCBLK

        @pl.when(base + wid * RPH < rows_needed)
        def _():
            pltpu.sync_copy(perm_hbm.at[pl.ds(base + wid * RPH, RPH)], idxv)
            wb = []
            for c in range(NCH_H):
                xb = xb0 if c % 2 == 0 else xb1
                ab = ab0 if c % 2 == 0 else ab1
                sem = sem0 if c % 2 == 0 else sem1
                if c >= 2:
                    wb[2 * (c - 2)].wait()
                    wb[2 * (c - 2) + 1].wait()
                idx_c = idxv.at[pl.ds(c * CH, CH)]
                hx = pltpu.async_copy(x_hbm.at[idx_c], xb, sem)
                ha = pltpu.async_copy(act_hbm.at[idx_c], ab, sem)
                hx.wait()
                ha.wait()
                row0 = wid * RPH + c * CH
                wb.append(pltpu.async_copy(xb, xg_hbm.at[pl.ds(row0, CH)], sem))
                wb.append(
                    pltpu.async_copy(ab, actg_hbm.at[pl.ds(row0, CH)], sem))
            for hdl in wb[-4:]:
                hdl.wait()

    return _g


_sc_gather_a = _make_gather(0)
_sc_gather_b = _make_gather(NH)


@functools.partial(
    pl.kernel, mesh=_MESH,
    out_type=[jax.ShapeDtypeStruct((N,), jnp.float32),
              jax.ShapeDtypeStruct((N,), jnp.float32)],
    scratch_types=[
        pltpu.VMEM((RPT,), jnp.int32),
        pltpu.VMEM((RPT,), jnp.float32),
        pltpu.VMEM((RPT,), jnp.float32),
        pltpu.VMEM((RPT,), jnp.float32),
        pltpu.VMEM((RPT,), jnp.float32),
        pltpu.VMEM((L,), jnp.int32),
        pltpu.VMEM_SHARED((N,), jnp.float32),
        pltpu.VMEM_SHARED((N,), jnp.float32),
    ],
)
def _sc_scatter(perm_hbm, valga_hbm, valgb_hbm, logpga_hbm, logpgb_hbm,
                kq_hbm, val_hbm, logp_hbm,
                pvm, vvm, lvm, vbuf, lbuf, kvm, sval, slogp):
    sid = lax.axis_index("s")
    cid = lax.axis_index("c")
    pltpu.sync_copy(kq_hbm, kvm)
    k_vec = kvm[...]
    iota = lax.iota(jnp.int32, L)
    for w in (2 * sid, 2 * sid + 1):
        pltpu.sync_copy(perm_hbm.at[pl.ds(w * RPT, RPT)], pvm)

        @pl.when(sid < NS // 2)
        def _():
            pltpu.sync_copy(valga_hbm.at[pl.ds(w * RPT, RPT)], vvm)
            pltpu.sync_copy(logpga_hbm.at[pl.ds(w * RPT, RPT)], lvm)

        @pl.when(sid >= NS // 2)
        def _():
            pltpu.sync_copy(valgb_hbm.at[pl.ds(w * RPT - NH, RPT)], vvm)
            pltpu.sync_copy(logpgb_hbm.at[pl.ds(w * RPT - NH, RPT)], lvm)
        for j in range(JPT):
            gidx = (w * RPT + j * L) + iota
            live = _mask01(k_vec - gidx).astype(jnp.float32)
            vbuf[pl.ds(j * L, L)] = vvm[pl.ds(j * L, L)] * live
            lbuf[pl.ds(j * L, L)] = lvm[pl.ds(j * L, L)] * live
        pltpu.sync_copy(vbuf, sval.at[pvm])
        pltpu.sync_copy(lbuf, slogp.at[pvm])
    plsc.subcore_barrier()
    half = cid * (N // 2) + sid * RPT
    pltpu.sync_copy(sval.at[pl.ds(half, RPT)], vbuf)
    pltpu.sync_copy(vbuf, val_hbm.at[pl.ds(half, RPT)])
    pltpu.sync_copy(slogp.at[pl.ds(half, RPT)], lbuf)
    pltpu.sync_copy(lbuf, logp_hbm.at[pl.ds(half, RPT)])


def _tc_compact(s_ref, xg_ref, actg_ref, w1_ref, b1_ref, w2_ref, b2_ref,
                logstd_ref, valg_ref, logpg_ref, ent_ref):
    i = pl.program_id(0)
    log_std = logstd_ref[...]
    sum_log_std = jnp.sum(log_std)

    @pl.when(i < s_ref[0])
    def _():
        h = jnp.tanh(jnp.dot(xg_ref[...], w1_ref[...],
                             preferred_element_type=jnp.float32) + b1_ref[...])
        out2 = (jnp.dot(h, w2_ref[...], preferred_element_type=jnp.float32)
                + b2_ref[...])
        val = out2[:, 0:1]
        mu = out2[:, 1:1 + A]
        inv_std = jnp.exp(-log_std)
        diff = (actg_ref[:, 0:A] - mu) * inv_std
        valg_ref[...] = val.reshape(TCBLK)
        logpg_ref[...] = (-0.5 * jnp.sum(diff * diff, axis=-1)
                          - sum_log_std - 0.5 * A * _LOG2PI)

    @pl.when(i >= s_ref[0])
    def _():
        valg_ref[...] = jnp.zeros((TCBLK,), jnp.float32)
        logpg_ref[...] = jnp.zeros((TCBLK,), jnp.float32)

    @pl.when(i == 0)
    def _():
        ent_scalar = sum_log_std + 0.5 * A * (_LOG2PI + 1.0)
        ent_ref[...] = (s_ref[1].astype(jnp.float32)
                        * (ent_scalar / N)).reshape(1, 1)


def _clamped(i, s):
    return (jnp.maximum(jnp.minimum(i, s[0] - 1), 0), 0)


def _tc_half(s, xg, actg, W1, b1r, W2, b2, lsr):
    return pl.pallas_call(
        _tc_compact,
        grid_spec=pltpu.PrefetchScalarGridSpec(
            num_scalar_prefetch=1,
            grid=(NH // TCBLK,),
            in_specs=[
                pl.BlockSpec((TCBLK, D), _clamped),
                pl.BlockSpec((TCBLK, 2 * A), _clamped),
                pl.BlockSpec((D, H), lambda i, s: (0, 0)),
                pl.BlockSpec((1, H), lambda i, s: (0, 0)),
                pl.BlockSpec((H, 1 + A), lambda i, s: (0, 0)),
                pl.BlockSpec((1, 1 + A), lambda i, s: (0, 0)),
                pl.BlockSpec((1, A), lambda i, s: (0, 0)),
            ],
            out_specs=[
                pl.BlockSpec((TCBLK,), lambda i, s: (i,)),
                pl.BlockSpec((TCBLK,), lambda i, s: (i,)),
                pl.BlockSpec((1, 1), lambda i, s: (0, 0)),
            ],
        ),
        out_shape=[
            jax.ShapeDtypeStruct((NH,), jnp.float32),
            jax.ShapeDtypeStruct((NH,), jnp.float32),
            jax.ShapeDtypeStruct((1, 1), jnp.float32),
        ],
    )(s, xg, actg, W1, b1r, W2, b2, lsr)


def kernel(stage, x, act, W1, b1, Wv, bv, Wa, ba, log_std):
    stage_i = stage.astype(jnp.int32)
    counts = _sc_count(stage_i)
    perm, kout = _sc_partition(stage_i, counts)
    act128 = jnp.pad(act, ((0, 0), (0, A)))
    xga, actga = _sc_gather_a(x, act128, perm, kout)
    xgb, actgb = _sc_gather_b(x, act128, perm, kout)

    k_total = kout[0]
    nblk = (k_total + (TCBLK - 1)) // TCBLK
    nbh = NH // TCBLK
    s_a = jnp.stack([jnp.minimum(nblk, nbh), k_total])
    s_b = jnp.stack([jnp.clip(nblk - nbh, 0, nbh), k_total])

    W2 = jnp.concatenate([Wv, Wa], axis=1)
    b2 = jnp.concatenate([bv, ba]).reshape(1, 1 + A)
    b1r = b1.reshape(1, H)
    lsr = log_std.reshape(1, A)

    valga, logpga, ent = _tc_half(s_a, xga, actga, W1, b1r, W2, b2, lsr)
    valgb, logpgb, _ = _tc_half(s_b, xgb, actgb, W1, b1r, W2, b2, lsr)

    val1, logp1 = _sc_scatter(perm, valga, valgb, logpga, logpgb, kout)
    return (val1.reshape(N, 1), logp1.reshape(N, 1), ent[0, 0])
